# Initial kernel scaffold; baseline (speedup 1.0000x reference)
import jax
import jax.numpy as jnp
from jax.experimental import pallas as pl


def kernel(x, edge_index, edge_attr, batch, W1e, b1e, W2e, b2e, W1n, b1n, W2n, b2n, Wo1, bo1, Wo2, bo2, Wo3, bo3, Wo4, bo4):
    raise NotImplementedError("write your pallas kernel here")


# speedup vs baseline: 3.9078x; 3.9078x over previous
import functools

import jax
import jax.numpy as jnp
from jax import lax
from jax.experimental import pallas as pl
from jax.experimental.pallas import tpu as pltpu
from jax.experimental.pallas import tpu_sc as plsc

N = 100000
E = 3200000
H = 16
G = 256

NC = 2
NS = 16
NW = NC * NS
EPW = E // NW
B = 80
NB = EPW // B
RPT_STRIDE = 6248
RPT_LEN = 6280
ZR = 40
BN = 2000
NBLK = N // BN


def _edge_body(x_hbm, rows_hbm, cols_hbm, ea_hbm,
               w1_hbm, b1_hbm, w2_hbm, b2_hbm, out_hbm,
               row_v, col_v, ea_v, xr_v, xc_v, stage_v, zbuf, w_v,
               S_sh, sem0, sem1, semg0, semg1):
    c = lax.axis_index("c")
    s = lax.axis_index("s")
    wid = c * NS + s

    pltpu.sync_copy(w1_hbm, w_v.at[0:5, :])
    pltpu.sync_copy(b1_hbm, w_v.at[5, :])
    pltpu.sync_copy(w2_hbm, w_v.at[6:22, :])
    pltpu.sync_copy(b2_hbm, w_v.at[22, :])

    zero16 = jnp.zeros((16,), jnp.float32)

    def zb(i, carry):
        zbuf[i, :] = zero16
        return carry
    lax.fori_loop(0, ZR, zb, 0)

    def zs(i, carry):
        pltpu.sync_copy(zbuf, S_sh.at[pl.ds(s * RPT_STRIDE + i * ZR, ZR), :])
        return carry
    lax.fori_loop(0, RPT_LEN // ZR, zs, 0)
    plsc.subcore_barrier()

    def bfround(v):
        u = plsc.bitcast(v, jnp.int32)
        lsb = lax.shift_right_logical(u, 16) & 1
        r = (u + (32767 + lsb)) & (-65536)
        return plsc.bitcast(r, jnp.float32)

    iota = lax.iota(jnp.int32, 16)
    onehot = [(iota == j).astype(jnp.float32) for j in range(H)]

    def sread(r, j, rounded=False):
        vec = w_v[r, :]
        if rounded:
            vec = bfround(vec)
        return jnp.sum(vec * onehot[j])

    w1 = [[sread(k, j, rounded=(k >= 2)) for j in range(H)] for k in range(5)]
    b1 = [sread(5, j) for j in range(H)]
    w2 = [[sread(6 + k, j, rounded=True) for j in range(H)] for k in range(H)]
    b2 = [sread(22, j) for j in range(H)]
    sems = (sem0, sem1)
    gsems = (semg0, semg1)

    def compute_block(b):
        bfull = jnp.full((16,), b, jnp.int32)
        for t in range(5):
            off = t * 16
            xr = xr_v[b, pl.ds(off, 16)]
            xc = xc_v[b, pl.ds(off, 16)]
            eid = jnp.full((16,), off, jnp.int32) + iota
            ea0 = bfround(plsc.load_gather(ea_v, [bfull, eid, jnp.full((16,), 0, jnp.int32)]))
            ea1 = bfround(plsc.load_gather(ea_v, [bfull, eid, jnp.full((16,), 1, jnp.int32)]))
            ea2 = bfround(plsc.load_gather(ea_v, [bfull, eid, jnp.full((16,), 2, jnp.int32)]))
            relu1 = []
            for j in range(H):
                xpart = xr * w1[0][j] + xc * w1[1][j]
                eapart = ea0 * w1[2][j] + ea1 * w1[3][j] + ea2 * w1[4][j]
                z = (xpart + eapart) + b1[j]
                relu1.append(bfround(jnp.maximum(z, 0.0)))
            for j in range(H):
                y = relu1[0] * w2[0][j]
                for k in range(1, H):
                    y = y + relu1[k] * w2[k][j]
                y = y + b2[j]
                plsc.store_scatter(stage_v, [bfull, eid, jnp.full((16,), j, jnp.int32)], y)

    def body(g, carry):
        for b in range(2):
            blk = g * 2 + b
            base = wid * EPW + blk * B
            pltpu.sync_copy(rows_hbm.at[pl.ds(base, B)], row_v.at[b])
            pltpu.sync_copy(cols_hbm.at[pl.ds(base, B)], col_v.at[b])
            pltpu.sync_copy(ea_hbm.at[pl.ds(base, B), :], ea_v.at[b])
        gd = []
        for b in range(2):
            gd.append(pltpu.async_copy(x_hbm.at[row_v.at[b]], xr_v.at[b], gsems[b]))
            gd.append(pltpu.async_copy(x_hbm.at[col_v.at[b]], xc_v.at[b], gsems[b]))
        descs = []
        for b in range(2):
            gd[2 * b].wait()
            gd[2 * b + 1].wait()
            compute_block(b)
            descs.append(pltpu.async_copy(
                stage_v.at[b], S_sh.at[col_v.at[b]], sems[b], add=True))
        for d in descs:
            d.wait()
        return carry

    lax.fori_loop(0, NB // 2, body, 0)
    plsc.subcore_barrier()
    pltpu.sync_copy(S_sh.at[pl.ds(s * RPT_STRIDE, RPT_LEN), :],
                    out_hbm.at[c, pl.ds(s * RPT_STRIDE, RPT_LEN), :])


_edge_kernel = functools.partial(
    pl.kernel,
    out_type=jax.ShapeDtypeStruct((NC, N, H), jnp.float32),
    mesh=plsc.VectorSubcoreMesh(core_axis_name="c", subcore_axis_name="s"),
    scratch_types=[
        pltpu.VMEM((2, B), jnp.int32),
        pltpu.VMEM((2, B), jnp.int32),
        pltpu.VMEM((2, B, 3), jnp.float32),
        pltpu.VMEM((2, B), jnp.float32),
        pltpu.VMEM((2, B), jnp.float32),
        pltpu.VMEM((2, B, H), jnp.float32),
        pltpu.VMEM((ZR, H), jnp.float32),
        pltpu.VMEM((23, H), jnp.float32),
        pltpu.VMEM_SHARED((N, H), jnp.float32),
        pltpu.SemaphoreType.DMA,
        pltpu.SemaphoreType.DMA,
        pltpu.SemaphoreType.DMA,
        pltpu.SemaphoreType.DMA,
    ],
    compiler_params=pltpu.CompilerParams(needs_layout_passes=False,
                                         use_tc_tiling_on_sc=False),
)(_edge_body)


def _node_body(x_ref, s0_ref, s1_ref, bt_ref, w1x_ref, w1s_ref, b1n_ref,
               w2n_ref, b2n_ref, wo1_ref, bo1_ref, wo2_ref, bo2_ref,
               wo3_ref, bo3_ref, wo4_ref, bo4_ref, out_ref, acc):
    i = pl.program_id(0)

    @pl.when(i == 0)
    def _():
        acc[...] = jnp.zeros_like(acc)

    hp = jax.lax.Precision.HIGHEST
    xb = x_ref[...]
    sb = s0_ref[...] + s1_ref[...]
    f32 = jnp.float32
    bf = jnp.bfloat16
    h1 = jnp.maximum(
        (jnp.dot(xb, w1x_ref[...], precision=hp)
         + jnp.dot(sb.astype(bf), w1s_ref[...].astype(bf),
                   preferred_element_type=f32)) + b1n_ref[...], 0.0)
    h = jnp.dot(h1.astype(bf), w2n_ref[...].astype(bf),
                preferred_element_type=f32) + b2n_ref[...]
    bt = bt_ref[0, 0, :]
    oh = (bt[:, None] == lax.broadcasted_iota(jnp.int32, (BN, G), 1)
          ).astype(jnp.float32)
    acc[...] += lax.dot_general(oh, h, (((0,), (0,)), ((), ())),
                                precision=hp,
                                preferred_element_type=jnp.float32)

    @pl.when(i == NBLK - 1)
    def _():
        def bdot(a, w):
            return jnp.dot(a.astype(bf), w.astype(bf),
                           preferred_element_type=f32)

        p = acc[...]
        o = jnp.maximum(bdot(p, wo1_ref[...]) + bo1_ref[...], 0.0)
        o = jnp.maximum(bdot(o, wo2_ref[...]) + bo2_ref[...], 0.0)
        o = jnp.maximum(bdot(o, wo3_ref[...]) + bo3_ref[...], 0.0)
        out_ref[...] = bdot(o, wo4_ref[...]) + bo4_ref[...]


def _full(shape):
    return pl.BlockSpec(shape, lambda i: (0,) * len(shape))


_node_kernel = pl.pallas_call(
    _node_body,
    grid=(NBLK,),
    in_specs=[
        pl.BlockSpec((BN, 1), lambda i: (i, 0)),
        pl.BlockSpec((BN, H), lambda i: (i, 0)),
        pl.BlockSpec((BN, H), lambda i: (i, 0)),
        pl.BlockSpec((1, 1, BN), lambda i: (i, 0, 0)),
        _full((1, H)), _full((H, H)), _full((1, H)),
        _full((H, H)), _full((1, H)),
        _full((H, H)), _full((1, H)),
        _full((H, H)), _full((1, H)),
        _full((H, H)), _full((1, H)),
        _full((H, 1)), _full((1, 1)),
    ],
    out_specs=pl.BlockSpec((G, 1), lambda i: (0, 0)),
    out_shape=jax.ShapeDtypeStruct((G, 1), jnp.float32),
    scratch_shapes=[pltpu.VMEM((G, H), jnp.float32)],
)


def kernel(x, edge_index, edge_attr, batch,
           W1e, b1e, W2e, b2e, W1n, b1n, W2n, b2n,
           Wo1, bo1, Wo2, bo2, Wo3, bo3, Wo4, bo4):
    xf = x.reshape(N)
    rows = edge_index[0]
    cols = edge_index[1]
    sp = _edge_kernel(xf, rows, cols, edge_attr, W1e, b1e, W2e, b2e)
    bt3 = batch.reshape(NBLK, 1, BN)
    return _node_kernel(
        x, sp[0], sp[1], bt3,
        W1n[0:1, :], W1n[1:, :], b1n.reshape(1, H),
        W2n, b2n.reshape(1, H),
        Wo1, bo1.reshape(1, H), Wo2, bo2.reshape(1, H),
        Wo3, bo3.reshape(1, H), Wo4, bo4.reshape(1, 1))
